# baseline (speedup 1.0000x reference)
import functools

import jax
import jax.numpy as jnp
from jax import lax
from jax.experimental import pallas as pl
from jax.experimental.pallas import tpu as pltpu
from jax.experimental.pallas import tpu_sc as plsc

NUM_SPK = 100000
EMBD_DIM = 64
BATCH = 16384

_NC = 2
_NS = 16
_NW = _NC * _NS
_BPW = BATCH // _NW
_CHUNK = 128
_NCHUNK = _BPW // _CHUNK

_mesh = plsc.VectorSubcoreMesh(core_axis_name="c", subcore_axis_name="s")


@functools.partial(
    pl.kernel,
    mesh=_mesh,
    out_type=jax.ShapeDtypeStruct((BATCH, EMBD_DIM), jnp.float32),
    scratch_types=[
        pltpu.VMEM((_NCHUNK, _CHUNK), jnp.int32),
        pltpu.VMEM((_BPW, EMBD_DIM), jnp.float32),
        pltpu.SemaphoreType.DMA,
    ],
    compiler_params=pltpu.CompilerParams(use_tc_tiling_on_sc=False),
)
def _gather_kernel(idx_hbm, table_hbm, out_hbm, idx_v, rows_v, sem):
    wid = lax.axis_index("s") * _NC + lax.axis_index("c")
    pltpu.sync_copy(idx_hbm.at[pl.ds(wid * _NCHUNK, _NCHUNK)], idx_v)
    copies = [
        pltpu.async_copy(
            table_hbm.at[idx_v.at[j]],
            rows_v.at[pl.ds(j * _CHUNK, _CHUNK)],
            sem,
        )
        for j in range(_NCHUNK)
    ]
    for c in copies:
        c.wait()
    pltpu.sync_copy(rows_v, out_hbm.at[pl.ds(wid * _BPW, _BPW)])


def kernel(spk_inds, table):
    idx2d = spk_inds.astype(jnp.int32).reshape(_NW * _NCHUNK, _CHUNK)
    return _gather_kernel(idx2d, table)


# speedup vs baseline: 1.0002x; 1.0002x over previous
import functools

import jax
import jax.numpy as jnp
from jax import lax
from jax.experimental import pallas as pl
from jax.experimental.pallas import tpu as pltpu
from jax.experimental.pallas import tpu_sc as plsc

NUM_SPK = 100000
EMBD_DIM = 64
BATCH = 16384

_NC = 2
_NS = 16
_NW = _NC * _NS
_BPW = BATCH // _NW
_CHUNK = 128
_NCHUNK = _BPW // _CHUNK

_mesh = plsc.VectorSubcoreMesh(core_axis_name="c", subcore_axis_name="s")


@functools.partial(
    pl.kernel,
    mesh=_mesh,
    out_type=jax.ShapeDtypeStruct((BATCH, EMBD_DIM), jnp.float32),
    scratch_types=[
        pltpu.VMEM((_BPW,), jnp.int32),
        pltpu.VMEM((_BPW, EMBD_DIM), jnp.float32),
        pltpu.SemaphoreType.DMA,
        pltpu.SemaphoreType.DMA,
    ],
    compiler_params=pltpu.CompilerParams(use_tc_tiling_on_sc=False),
)
def _gather_kernel(idx_hbm, table_hbm, out_hbm, idx_v, rows_v, sem_g, sem_w):
    wid = lax.axis_index("s") * _NC + lax.axis_index("c")
    base = wid * _BPW
    pltpu.sync_copy(idx_hbm.at[pl.ds(base, _BPW)], idx_v)
    gathers = [
        pltpu.async_copy(
            table_hbm.at[idx_v.at[pl.ds(j * _CHUNK, _CHUNK)]],
            rows_v.at[pl.ds(j * _CHUNK, _CHUNK)],
            sem_g,
        )
        for j in range(_NCHUNK)
    ]
    writes = []
    for j in range(_NCHUNK):
        gathers[j].wait()
        writes.append(
            pltpu.async_copy(
                rows_v.at[pl.ds(j * _CHUNK, _CHUNK)],
                out_hbm.at[pl.ds(base + j * _CHUNK, _CHUNK)],
                sem_w,
            )
        )
    for w in writes:
        w.wait()


def kernel(spk_inds, table):
    return _gather_kernel(spk_inds.astype(jnp.int32), table)


# speedup vs baseline: 1.1507x; 1.1505x over previous
import functools

import jax
import jax.numpy as jnp
from jax import lax
from jax.experimental import pallas as pl
from jax.experimental.pallas import tpu as pltpu
from jax.experimental.pallas import tpu_sc as plsc

NUM_SPK = 100000
EMBD_DIM = 64
BATCH = 16384

_NC = 2
_NS = 16
_NW = _NC * _NS
_BPW = BATCH // _NW
_LAG = 16

_mesh = plsc.VectorSubcoreMesh(core_axis_name="c", subcore_axis_name="s")


@functools.partial(
    pl.kernel,
    mesh=_mesh,
    out_type=jax.ShapeDtypeStruct((EMBD_DIM, BATCH), jnp.float32),
    scratch_types=[
        pltpu.VMEM((_BPW,), jnp.int32),
        pltpu.VMEM((_BPW, EMBD_DIM), jnp.float32),
        pltpu.VMEM((EMBD_DIM, _BPW), jnp.float32),
        pltpu.SemaphoreType.DMA,
        pltpu.SemaphoreType.DMA,
    ],
    compiler_params=pltpu.CompilerParams(
        use_tc_tiling_on_sc=True, needs_layout_passes=False
    ),
)
def _gather_kernel(
    idx_hbm, table_hbm, out_hbm, idx_v, rows_v, vout, sem_g, sem_w
):
    wid = lax.axis_index("s") * _NC + lax.axis_index("c")
    base = wid * _BPW
    pltpu.sync_copy(idx_hbm.at[pl.ds(base, _BPW)], idx_v)

    def _row_body(g, carry):
        svec = idx_v[pl.ds(g * 16, 16)]
        for j in range(16):
            s = svec[j]
            pltpu.async_copy(
                table_hbm.at[pl.ds(s, 1)],
                rows_v.at[pl.ds(g * 16 + j, 1)],
                sem_g,
            )

        @pl.when(g >= 1)
        def _():
            for _ in range(16):
                pltpu.make_async_copy(
                    table_hbm.at[pl.ds(0, 1)], rows_v.at[pl.ds(0, 1)], sem_g
                ).wait()

        return carry

    lax.fori_loop(0, _BPW // 16, _row_body, 0)
    for _ in range(16):
        pltpu.make_async_copy(
            table_hbm.at[pl.ds(0, 1)], rows_v.at[pl.ds(0, 1)], sem_g
        ).wait()

    lanes = lax.iota(jnp.int32, 16)

    def _tr_body(b, carry):
        bvec = jnp.full((16,), b, jnp.int32)
        for k in range(EMBD_DIM // 16):
            piece = rows_v[b, pl.ds(k * 16, 16)]
            plsc.store_scatter(vout, [k * 16 + lanes, bvec], piece)
        return carry

    lax.fori_loop(0, _BPW, _tr_body, 0)

    pltpu.sync_copy(vout, out_hbm.at[:, pl.ds(base, _BPW)])


def kernel(spk_inds, table):
    out_t = _gather_kernel(spk_inds.astype(jnp.int32), table)
    return out_t.T


# speedup vs baseline: 1.1535x; 1.0024x over previous
import functools

import jax
import jax.numpy as jnp
from jax import lax
from jax.experimental import pallas as pl
from jax.experimental.pallas import tpu as pltpu
from jax.experimental.pallas import tpu_sc as plsc

NUM_SPK = 100000
EMBD_DIM = 64
BATCH = 16384

_NC = 2
_NS = 16
_NW = _NC * _NS
_BPW = BATCH // _NW
_LAG = 16

_mesh = plsc.VectorSubcoreMesh(core_axis_name="c", subcore_axis_name="s")


@functools.partial(
    pl.kernel,
    mesh=_mesh,
    out_type=jax.ShapeDtypeStruct((EMBD_DIM, BATCH), jnp.float32),
    scratch_types=[
        pltpu.VMEM((_BPW,), jnp.int32),
        pltpu.VMEM((_BPW, EMBD_DIM), jnp.float32),
        pltpu.VMEM((EMBD_DIM, _BPW + 17), jnp.float32),
        pltpu.SemaphoreType.DMA,
        pltpu.SemaphoreType.DMA,
    ],
    compiler_params=pltpu.CompilerParams(
        use_tc_tiling_on_sc=True, needs_layout_passes=False
    ),
)
def _gather_kernel(
    idx_hbm, table_hbm, out_hbm, idx_v, rows_v, vout, sem_g, sem_w
):
    wid = lax.axis_index("s") * _NC + lax.axis_index("c")
    base = wid * _BPW
    pltpu.sync_copy(idx_hbm.at[pl.ds(base, _BPW)], idx_v)

    def _row_body(g, carry):
        svec = idx_v[pl.ds(g * 16, 16)]
        for j in range(16):
            s = svec[j]
            pltpu.async_copy(
                table_hbm.at[pl.ds(s, 1)],
                rows_v.at[pl.ds(g * 16 + j, 1)],
                sem_g,
            )

        @pl.when(g >= 1)
        def _():
            for _ in range(16):
                pltpu.make_async_copy(
                    table_hbm.at[pl.ds(0, 1)], rows_v.at[pl.ds(0, 1)], sem_g
                ).wait()

        return carry

    lax.fori_loop(0, _BPW // 16, _row_body, 0)
    for _ in range(16):
        pltpu.make_async_copy(
            table_hbm.at[pl.ds(0, 1)], rows_v.at[pl.ds(0, 1)], sem_g
        ).wait()

    lanes = lax.iota(jnp.int32, 16)

    def _tr_body(b, carry):
        bvec = jnp.full((16,), b, jnp.int32)
        for k in range(EMBD_DIM // 16):
            piece = rows_v[b, pl.ds(k * 16, 16)]
            plsc.store_scatter(vout, [k * 16 + lanes, bvec], piece)
        return carry

    lax.fori_loop(0, _BPW, _tr_body, 0)

    pltpu.sync_copy(vout.at[:, pl.ds(0, _BPW)], out_hbm.at[:, pl.ds(base, _BPW)])


def kernel(spk_inds, table):
    out_t = _gather_kernel(spk_inds.astype(jnp.int32), table)
    return out_t.T
